# baseline (speedup 1.0000x reference)
import functools

import jax
import jax.numpy as jnp
from jax import lax
from jax.experimental import pallas as pl
from jax.experimental.pallas import tpu as pltpu
from jax.experimental.pallas import tpu_sc as plsc

_H = 768
_I = 256
_E = 64
_K = 2
_SCALE = 1.0
_BT = 128
_TMAX = 96
_TM = 256


def _shared_router_body(x_ref, wg_ref, wu_ref, wd_ref, wr_ref,
                        base_ref, idx_ref, w_ref):
    xt = x_ref[...]
    xb = xt.astype(jnp.bfloat16)
    g = jnp.dot(xb, wg_ref[...].astype(jnp.bfloat16),
                preferred_element_type=jnp.float32)
    u = jnp.dot(xb, wu_ref[...].astype(jnp.bfloat16),
                preferred_element_type=jnp.float32)
    h = (jax.nn.sigmoid(g) * u).astype(jnp.bfloat16)
    so = jnp.dot(h, wd_ref[...].astype(jnp.bfloat16),
                 preferred_element_type=jnp.float32)
    base_ref[...] = xt + so
    logits = jnp.dot(xt, wr_ref[...], preferred_element_type=jnp.float32)
    m = jnp.max(logits, axis=-1, keepdims=True)
    e = jnp.exp(logits - m)
    p = e / jnp.sum(e, axis=-1, keepdims=True)
    i1 = jnp.argmax(p, axis=-1).astype(jnp.int32)
    m1 = jnp.max(p, axis=-1)
    lane = lax.broadcasted_iota(jnp.int32, p.shape, 1)
    p2 = jnp.where(lane == i1[:, None], -1.0, p)
    i2 = jnp.argmax(p2, axis=-1).astype(jnp.int32)
    m2 = jnp.max(p2, axis=-1)
    idx_ref[...] = jnp.stack([i1, i2], axis=0)
    w_ref[...] = jnp.stack([m1, m2], axis=0)


def _shared_router(x2d, Wg_s, Wu_s, Wd_s, Wr):
    s = x2d.shape[0]
    return pl.pallas_call(
        _shared_router_body,
        grid=(s // _TM,),
        in_specs=[
            pl.BlockSpec((_TM, _H), lambda i: (i, 0)),
            pl.BlockSpec((_H, _I), lambda i: (0, 0)),
            pl.BlockSpec((_H, _I), lambda i: (0, 0)),
            pl.BlockSpec((_I, _H), lambda i: (0, 0)),
            pl.BlockSpec((_H, _E), lambda i: (0, 0)),
        ],
        out_specs=[
            pl.BlockSpec((_TM, _H), lambda i: (i, 0)),
            pl.BlockSpec((_K, _TM), lambda i: (0, i)),
            pl.BlockSpec((_K, _TM), lambda i: (0, i)),
        ],
        out_shape=[
            jax.ShapeDtypeStruct((s, _H), jnp.float32),
            jax.ShapeDtypeStruct((_K, s), jnp.int32),
            jax.ShapeDtypeStruct((_K, s), jnp.float32),
        ],
    )(x2d, Wg_s, Wu_s, Wd_s, Wr)


def _group_mlp_body(te_ref, tv_ref, xs_ref, wg_ref, wu_ref, wd_ref, out_ref):
    del te_ref
    t = pl.program_id(0)

    @pl.when(tv_ref[t] == 1)
    def _():
        xt = xs_ref[...].astype(jnp.bfloat16)
        g = jnp.dot(xt, wg_ref[0].astype(jnp.bfloat16),
                    preferred_element_type=jnp.float32)
        u = jnp.dot(xt, wu_ref[0].astype(jnp.bfloat16),
                    preferred_element_type=jnp.float32)
        h = (jax.nn.sigmoid(g) * u).astype(jnp.bfloat16)
        out_ref[...] = jnp.dot(h, wd_ref[0].astype(jnp.bfloat16),
                               preferred_element_type=jnp.float32)


def _group_mlp(xs, Wg, Wu, Wd, tile_eid, tile_valid):
    grid_spec = pltpu.PrefetchScalarGridSpec(
        num_scalar_prefetch=2,
        grid=(_TMAX,),
        in_specs=[
            pl.BlockSpec((_BT, _H), lambda t, te, tv: (t * tv[t], 0)),
            pl.BlockSpec((1, _H, _I), lambda t, te, tv: (te[t], 0, 0)),
            pl.BlockSpec((1, _H, _I), lambda t, te, tv: (te[t], 0, 0)),
            pl.BlockSpec((1, _I, _H), lambda t, te, tv: (te[t], 0, 0)),
        ],
        out_specs=pl.BlockSpec(
            (_BT, _H),
            lambda t, te, tv: (t * tv[t] + (1 - tv[t]) * _TMAX, 0)),
    )
    return pl.pallas_call(
        _group_mlp_body,
        grid_spec=grid_spec,
        out_shape=jax.ShapeDtypeStruct(((_TMAX + 1) * _BT, _H), jnp.float32),
    )(tile_eid, tile_valid, xs, Wg, Wu, Wd)


def _sc_gather_rows(table, idx, chunk):
    b = idx.shape[0]
    d = table.shape[1]
    nw = 32
    b_per_w = b // nw
    mesh = plsc.VectorSubcoreMesh(core_axis_name="c", subcore_axis_name="s",
                                  num_cores=2, num_subcores=16)

    @functools.partial(
        pl.kernel,
        out_type=jax.ShapeDtypeStruct((b, d), table.dtype),
        mesh=mesh,
        scratch_types=[
            pltpu.VMEM((chunk,), jnp.int32),
            pltpu.VMEM((chunk, d), table.dtype),
            pltpu.SemaphoreType.DMA,
        ],
    )
    def k(table_hbm, idx_hbm, out_hbm, idx_v, rows_v, sem):
        wid = lax.axis_index("s") * 2 + lax.axis_index("c")
        base = wid * b_per_w

        @pl.loop(0, b_per_w, step=chunk)
        def _(off):
            pltpu.sync_copy(idx_hbm.at[pl.ds(base + off, chunk)], idx_v)
            pltpu.async_copy(table_hbm.at[idx_v], rows_v, sem).wait()
            pltpu.sync_copy(rows_v, out_hbm.at[pl.ds(base + off, chunk)])

    return k(table, idx)


def _sc_scatter_rows(table, dst_idx, out_rows, chunk):
    s, d = table.shape
    n = dst_idx.shape[0]
    nw = 32
    b_per_w = n // nw
    mesh = plsc.VectorSubcoreMesh(core_axis_name="c", subcore_axis_name="s",
                                  num_cores=2, num_subcores=16)

    @functools.partial(
        pl.kernel,
        out_type=jax.ShapeDtypeStruct((out_rows, d), table.dtype),
        mesh=mesh,
        scratch_types=[
            pltpu.VMEM((chunk,), jnp.int32),
            pltpu.VMEM((chunk, d), table.dtype),
            pltpu.SemaphoreType.DMA,
        ],
    )
    def k(table_hbm, idx_hbm, out_hbm, idx_v, rows_v, sem):
        wid = lax.axis_index("s") * 2 + lax.axis_index("c")
        base = wid * b_per_w

        @pl.loop(0, b_per_w, step=chunk)
        def _(off):
            i0 = base + off
            src = lax.rem(i0, s)
            pltpu.sync_copy(idx_hbm.at[pl.ds(i0, chunk)], idx_v)
            pltpu.sync_copy(table_hbm.at[pl.ds(src, chunk)], rows_v)
            pltpu.async_copy(rows_v, out_hbm.at[idx_v], sem).wait()

    return k(table, dst_idx)


def _combine_body(base_ref, p0_ref, p1_ref, w_ref, out_ref):
    w0 = w_ref[0, 0, :][:, None]
    w1 = w_ref[1, 0, :][:, None]
    out_ref[...] = (base_ref[...]
                    + w0 * p0_ref[...].astype(jnp.float32)
                    + w1 * p1_ref[...].astype(jnp.float32))


def _combine(base, picked, w2s):
    s = base.shape[0]
    return pl.pallas_call(
        _combine_body,
        grid=(s // _TM,),
        in_specs=[
            pl.BlockSpec((_TM, _H), lambda i: (i, 0)),
            pl.BlockSpec((_TM, _H), lambda i: (i, 0)),
            pl.BlockSpec((_TM, _H), lambda i, _o=s // _TM: (_o + i, 0)),
            pl.BlockSpec((_K, 1, _TM), lambda i: (0, 0, i)),
        ],
        out_specs=pl.BlockSpec((_TM, _H), lambda i: (i, 0)),
        out_shape=jax.ShapeDtypeStruct((s, _H), jnp.float32),
    )(base, picked, picked, w2s.reshape(_K, 1, s))


def _plan_body(idx_ref, dest_ref, te_ref, tv_ref):
    nb = idx_ref.shape[0]
    eid = idx_ref[...]
    lane = lax.broadcasted_iota(jnp.int32, (nb, _BT, _E), 2)
    oh3 = (eid[:, :, None] == lane).astype(jnp.float32)
    r_i = lax.broadcasted_iota(jnp.int32, (_BT, _BT), 0)
    c_i = lax.broadcasted_iota(jnp.int32, (_BT, _BT), 1)
    ltri = (c_i < r_i).astype(jnp.float32)
    intra = jnp.stack([
        jnp.dot(ltri, oh3[b], preferred_element_type=jnp.float32)
        for b in range(nb)
    ])
    btot = jnp.sum(oh3, axis=1)
    rb_i = lax.broadcasted_iota(jnp.int32, (nb, nb), 0)
    cb_i = lax.broadcasted_iota(jnp.int32, (nb, nb), 1)
    lb = (cb_i > rb_i).astype(jnp.float32)
    boff = jnp.dot(lb.T, btot, preferred_element_type=jnp.float32)
    counts = jnp.sum(btot, axis=0).reshape(1, _E)
    tiles = jnp.floor((counts + (_BT - 1.0)) * (1.0 / _BT))
    re_i = lax.broadcasted_iota(jnp.int32, (_E, _E), 0)
    ce_i = lax.broadcasted_iota(jnp.int32, (_E, _E), 1)
    su = (re_i < ce_i).astype(jnp.float32)
    ts = jnp.dot(tiles, su, preferred_element_type=jnp.float32)
    base_f = ts * float(_BT)
    rank3 = intra + boff[:, None, :] + base_f[0][None, None, :]
    dest_ref[...] = jnp.sum(oh3 * rank3, axis=2).astype(jnp.int32)
    ts_i = ts.astype(jnp.int32)
    t_i = lax.broadcasted_iota(jnp.int32, (128, _E), 0)
    te = jnp.sum((ts_i[0][None, :] <= t_i).astype(jnp.int32), axis=1) - 1
    te_ref[...] = te.reshape(1, 128)
    total = jnp.sum(tiles).astype(jnp.int32)
    tv = (lax.broadcasted_iota(jnp.int32, (1, 128), 1) < total)
    tv_ref[...] = tv.astype(jnp.int32)


def _plan(top_idx2s):
    n = top_idx2s.size
    nb = n // _BT
    idx3 = top_idx2s.reshape(nb, _BT)
    dest, te, tv = pl.pallas_call(
        _plan_body,
        out_shape=[
            jax.ShapeDtypeStruct((nb, _BT), jnp.int32),
            jax.ShapeDtypeStruct((1, 128), jnp.int32),
            jax.ShapeDtypeStruct((1, 128), jnp.int32),
        ],
    )(idx3)
    return te.reshape(128)[:_TMAX], tv.reshape(128)[:_TMAX], dest.reshape(n)


def kernel(x, Wg_s, Wu_s, Wd_s, Wg, Wu, Wd, Wr):
    b, s, h = x.shape
    flat = x.reshape(s, h)
    base, top_idx2s, top_w2s = _shared_router(flat, Wg_s, Wu_s, Wd_s, Wr)
    tile_eid, tile_valid, dest = _plan(top_idx2s)
    xs = _sc_scatter_rows(flat, dest, _TMAX * _BT, 64)
    ys = _group_mlp(xs, Wg, Wu, Wd, tile_eid, tile_valid)
    picked = _sc_gather_rows(ys, dest, 64)
    return _combine(base, picked, top_w2s * _SCALE).reshape(b, s, h)


# speedup vs baseline: 1.0063x; 1.0063x over previous
import functools

import jax
import jax.numpy as jnp
from jax import lax
from jax.experimental import pallas as pl
from jax.experimental.pallas import tpu as pltpu
from jax.experimental.pallas import tpu_sc as plsc

_H = 768
_I = 256
_E = 64
_K = 2
_SCALE = 1.0
_BT = 128
_TMAX = 96
_TM = 256


def _shared_router_body(x_ref, wg_ref, wu_ref, wd_ref, wr_ref,
                        base_ref, idx_ref, w_ref):
    xt = x_ref[...]
    xb = xt.astype(jnp.bfloat16)
    g = jnp.dot(xb, wg_ref[...].astype(jnp.bfloat16),
                preferred_element_type=jnp.float32)
    u = jnp.dot(xb, wu_ref[...].astype(jnp.bfloat16),
                preferred_element_type=jnp.float32)
    h = (jax.nn.sigmoid(g) * u).astype(jnp.bfloat16)
    so = jnp.dot(h, wd_ref[...].astype(jnp.bfloat16),
                 preferred_element_type=jnp.float32)
    base_ref[...] = xt + so
    logits = jnp.dot(xt, wr_ref[...], preferred_element_type=jnp.float32)
    m = jnp.max(logits, axis=-1, keepdims=True)
    e = jnp.exp(logits - m)
    p = e / jnp.sum(e, axis=-1, keepdims=True)
    i1 = jnp.argmax(p, axis=-1).astype(jnp.int32)
    m1 = jnp.max(p, axis=-1)
    lane = lax.broadcasted_iota(jnp.int32, p.shape, 1)
    p2 = jnp.where(lane == i1[:, None], -1.0, p)
    i2 = jnp.argmax(p2, axis=-1).astype(jnp.int32)
    m2 = jnp.max(p2, axis=-1)
    idx_ref[...] = jnp.stack([i1, i2], axis=0)
    w_ref[...] = jnp.stack([m1, m2], axis=0)


def _shared_router(x2d, Wg_s, Wu_s, Wd_s, Wr):
    s = x2d.shape[0]
    return pl.pallas_call(
        _shared_router_body,
        grid=(s // _TM,),
        in_specs=[
            pl.BlockSpec((_TM, _H), lambda i: (i, 0)),
            pl.BlockSpec((_H, _I), lambda i: (0, 0)),
            pl.BlockSpec((_H, _I), lambda i: (0, 0)),
            pl.BlockSpec((_I, _H), lambda i: (0, 0)),
            pl.BlockSpec((_H, _E), lambda i: (0, 0)),
        ],
        out_specs=[
            pl.BlockSpec((_TM, _H), lambda i: (i, 0)),
            pl.BlockSpec((_K, _TM), lambda i: (0, i)),
            pl.BlockSpec((_K, _TM), lambda i: (0, i)),
        ],
        out_shape=[
            jax.ShapeDtypeStruct((s, _H), jnp.float32),
            jax.ShapeDtypeStruct((_K, s), jnp.int32),
            jax.ShapeDtypeStruct((_K, s), jnp.float32),
        ],
    )(x2d, Wg_s, Wu_s, Wd_s, Wr)


def _group_mlp_body(te_ref, tv_ref, xs_ref, wg_ref, wu_ref, wd_ref, out_ref):
    del te_ref
    t = pl.program_id(0)

    @pl.when(tv_ref[0, t] == 1)
    def _():
        xt = xs_ref[...].astype(jnp.bfloat16)
        g = jnp.dot(xt, wg_ref[0].astype(jnp.bfloat16),
                    preferred_element_type=jnp.float32)
        u = jnp.dot(xt, wu_ref[0].astype(jnp.bfloat16),
                    preferred_element_type=jnp.float32)
        h = (jax.nn.sigmoid(g) * u).astype(jnp.bfloat16)
        out_ref[...] = jnp.dot(h, wd_ref[0].astype(jnp.bfloat16),
                               preferred_element_type=jnp.float32)


def _group_mlp(xs, Wg, Wu, Wd, tile_eid, tile_valid):
    grid_spec = pltpu.PrefetchScalarGridSpec(
        num_scalar_prefetch=2,
        grid=(_TMAX,),
        in_specs=[
            pl.BlockSpec((_BT, _H), lambda t, te, tv: (t * tv[0, t], 0)),
            pl.BlockSpec((1, _H, _I), lambda t, te, tv: (te[0, t], 0, 0)),
            pl.BlockSpec((1, _H, _I), lambda t, te, tv: (te[0, t], 0, 0)),
            pl.BlockSpec((1, _I, _H), lambda t, te, tv: (te[0, t], 0, 0)),
        ],
        out_specs=pl.BlockSpec(
            (_BT, _H),
            lambda t, te, tv: (t * tv[0, t] + (1 - tv[0, t]) * _TMAX, 0)),
    )
    return pl.pallas_call(
        _group_mlp_body,
        grid_spec=grid_spec,
        out_shape=jax.ShapeDtypeStruct(((_TMAX + 1) * _BT, _H), jnp.float32),
    )(tile_eid, tile_valid, xs, Wg, Wu, Wd)


def _sc_gather_rows(table, idx2, chunk):
    nw, b_per_w = idx2.shape
    b = nw * b_per_w
    d = table.shape[1]
    mesh = plsc.VectorSubcoreMesh(core_axis_name="c", subcore_axis_name="s",
                                  num_cores=2, num_subcores=16)

    @functools.partial(
        pl.kernel,
        out_type=jax.ShapeDtypeStruct((b, d), table.dtype),
        mesh=mesh,
        scratch_types=[
            pltpu.VMEM((chunk,), jnp.int32),
            pltpu.VMEM((chunk, d), table.dtype),
            pltpu.SemaphoreType.DMA,
        ],
    )
    def k(table_hbm, idx_hbm, out_hbm, idx_v, rows_v, sem):
        wid = lax.axis_index("s") * 2 + lax.axis_index("c")
        base = wid * b_per_w

        @pl.loop(0, b_per_w, step=chunk)
        def _(off):
            pltpu.sync_copy(idx_hbm.at[wid, pl.ds(off, chunk)], idx_v)
            pltpu.async_copy(table_hbm.at[idx_v], rows_v, sem).wait()
            pltpu.sync_copy(rows_v, out_hbm.at[pl.ds(base + off, chunk)])

    return k(table, idx2)


def _sc_scatter_rows(table, dst_idx2, out_rows, chunk):
    s, d = table.shape
    nw, b_per_w = dst_idx2.shape
    mesh = plsc.VectorSubcoreMesh(core_axis_name="c", subcore_axis_name="s",
                                  num_cores=2, num_subcores=16)

    @functools.partial(
        pl.kernel,
        out_type=jax.ShapeDtypeStruct((out_rows, d), table.dtype),
        mesh=mesh,
        scratch_types=[
            pltpu.VMEM((chunk,), jnp.int32),
            pltpu.VMEM((chunk, d), table.dtype),
            pltpu.SemaphoreType.DMA,
        ],
    )
    def k(table_hbm, idx_hbm, out_hbm, idx_v, rows_v, sem):
        wid = lax.axis_index("s") * 2 + lax.axis_index("c")
        base = wid * b_per_w

        @pl.loop(0, b_per_w, step=chunk)
        def _(off):
            src = lax.rem(base + off, s)
            pltpu.sync_copy(idx_hbm.at[wid, pl.ds(off, chunk)], idx_v)
            pltpu.sync_copy(table_hbm.at[pl.ds(src, chunk)], rows_v)
            pltpu.async_copy(rows_v, out_hbm.at[idx_v], sem).wait()

    return k(table, dst_idx2)


def _combine_body(base_ref, p0_ref, p1_ref, w_ref, out_ref):
    w0 = w_ref[0, 0, :][:, None]
    w1 = w_ref[1, 0, :][:, None]
    out_ref[...] = (base_ref[...]
                    + w0 * p0_ref[...].astype(jnp.float32)
                    + w1 * p1_ref[...].astype(jnp.float32))


def _combine(base, picked, w2s):
    s = base.shape[0]
    return pl.pallas_call(
        _combine_body,
        grid=(s // _TM,),
        in_specs=[
            pl.BlockSpec((_TM, _H), lambda i: (i, 0)),
            pl.BlockSpec((_TM, _H), lambda i: (i, 0)),
            pl.BlockSpec((_TM, _H), lambda i, _o=s // _TM: (_o + i, 0)),
            pl.BlockSpec((_K, 1, _TM), lambda i: (0, 0, i)),
        ],
        out_specs=pl.BlockSpec((_TM, _H), lambda i: (i, 0)),
        out_shape=jax.ShapeDtypeStruct((s, _H), jnp.float32),
    )(base, picked, picked, w2s.reshape(_K, 1, s))


def _plan_body(idx_ref, dest_ref, te_ref, tv_ref):
    k_, s_ = idx_ref.shape
    nb = (k_ * s_) // _BT
    eid = idx_ref[...]
    lane = lax.broadcasted_iota(jnp.int32, (k_, s_, _E), 2)
    oh3 = (eid[:, :, None] == lane).astype(jnp.float32).reshape(nb, _BT, _E)
    r_i = lax.broadcasted_iota(jnp.int32, (_BT, _BT), 0)
    c_i = lax.broadcasted_iota(jnp.int32, (_BT, _BT), 1)
    ltri = (c_i < r_i).astype(jnp.float32)
    intra = jnp.stack([
        jnp.dot(ltri, oh3[b], preferred_element_type=jnp.float32)
        for b in range(nb)
    ])
    btot = jnp.sum(oh3, axis=1)
    rb_i = lax.broadcasted_iota(jnp.int32, (nb, nb), 0)
    cb_i = lax.broadcasted_iota(jnp.int32, (nb, nb), 1)
    lb = (cb_i > rb_i).astype(jnp.float32)
    boff = jnp.dot(lb.T, btot, preferred_element_type=jnp.float32)
    counts = jnp.sum(btot, axis=0).reshape(1, _E)
    tiles = jnp.floor((counts + (_BT - 1.0)) * (1.0 / _BT))
    re_i = lax.broadcasted_iota(jnp.int32, (_E, _E), 0)
    ce_i = lax.broadcasted_iota(jnp.int32, (_E, _E), 1)
    su = (re_i < ce_i).astype(jnp.float32)
    ts = jnp.dot(tiles, su, preferred_element_type=jnp.float32)
    base_f = ts * float(_BT)
    rank3 = intra + boff[:, None, :] + base_f[0][None, None, :]
    dest_ref[...] = jnp.sum(oh3 * rank3, axis=2).astype(jnp.int32)
    ts_i = ts.astype(jnp.int32)
    t_i = lax.broadcasted_iota(jnp.int32, (128, _E), 0)
    te = jnp.sum((ts_i[0][None, :] <= t_i).astype(jnp.int32), axis=1) - 1
    te_ref[...] = te.reshape(1, 128)
    total = jnp.sum(tiles).astype(jnp.int32)
    tv = (lax.broadcasted_iota(jnp.int32, (1, 128), 1) < total)
    tv_ref[...] = tv.astype(jnp.int32)


def _plan(top_idx2s):
    n = top_idx2s.size
    nb = n // _BT
    dest, te, tv = pl.pallas_call(
        _plan_body,
        out_shape=[
            jax.ShapeDtypeStruct((nb, _BT), jnp.int32),
            jax.ShapeDtypeStruct((1, 128), jnp.int32),
            jax.ShapeDtypeStruct((1, 128), jnp.int32),
        ],
    )(top_idx2s)
    return te, tv, dest


def kernel(x, Wg_s, Wu_s, Wd_s, Wg, Wu, Wd, Wr):
    b, s, h = x.shape
    flat = x.reshape(s, h)
    base, top_idx2s, top_w2s = _shared_router(flat, Wg_s, Wu_s, Wd_s, Wr)
    tile_eid, tile_valid, dest = _plan(top_idx2s)
    xs = _sc_scatter_rows(flat, dest, _TMAX * _BT, 64)
    ys = _group_mlp(xs, Wg, Wu, Wd, tile_eid, tile_valid)
    picked = _sc_gather_rows(ys, dest, 64)
    return _combine(base, picked, top_w2s * _SCALE).reshape(b, s, h)


# speedup vs baseline: 1.0993x; 1.0925x over previous
import functools

import jax
import jax.numpy as jnp
from jax import lax
from jax.experimental import pallas as pl
from jax.experimental.pallas import tpu as pltpu
from jax.experimental.pallas import tpu_sc as plsc

_H = 768
_I = 256
_E = 64
_K = 2
_SCALE = 1.0
_BT = 128
_TMAX = 96
_TM = 256
_HP = _H // 2


def _pack_bf16(a):
    ab = a.astype(jnp.bfloat16)
    lo = lax.convert_element_type(
        lax.bitcast_convert_type(ab[..., :_HP], jnp.int16), jnp.int32)
    hi = lax.convert_element_type(
        lax.bitcast_convert_type(ab[..., _HP:], jnp.int16), jnp.int32)
    return (lo & 0xFFFF) | (hi << 16)


def _unpack_bf16(p):
    lo = lax.bitcast_convert_type(
        lax.convert_element_type(p, jnp.int16), jnp.bfloat16)
    hi = lax.bitcast_convert_type(
        lax.convert_element_type(
            lax.shift_right_logical(p, 16), jnp.int16), jnp.bfloat16)
    return jnp.concatenate([lo, hi], axis=-1)


def _shared_router_body(x_ref, wg_ref, wu_ref, wd_ref, wr_ref,
                        base_ref, idx_ref, w_ref, xp_ref):
    xt = x_ref[...]
    xb = xt.astype(jnp.bfloat16)
    g = jnp.dot(xb, wg_ref[...].astype(jnp.bfloat16),
                preferred_element_type=jnp.float32)
    u = jnp.dot(xb, wu_ref[...].astype(jnp.bfloat16),
                preferred_element_type=jnp.float32)
    h = (jax.nn.sigmoid(g) * u).astype(jnp.bfloat16)
    so = jnp.dot(h, wd_ref[...].astype(jnp.bfloat16),
                 preferred_element_type=jnp.float32)
    base_ref[...] = xt + so
    logits = jnp.dot(xt, wr_ref[...], preferred_element_type=jnp.float32)
    m = jnp.max(logits, axis=-1, keepdims=True)
    e = jnp.exp(logits - m)
    p = e / jnp.sum(e, axis=-1, keepdims=True)
    i1 = jnp.argmax(p, axis=-1).astype(jnp.int32)
    m1 = jnp.max(p, axis=-1)
    lane = lax.broadcasted_iota(jnp.int32, p.shape, 1)
    p2 = jnp.where(lane == i1[:, None], -1.0, p)
    i2 = jnp.argmax(p2, axis=-1).astype(jnp.int32)
    m2 = jnp.max(p2, axis=-1)
    idx_ref[...] = jnp.stack([i1, i2], axis=0)
    w_ref[...] = jnp.stack([m1, m2], axis=0)
    xp_ref[...] = _pack_bf16(xb)


def _shared_router(x2d, Wg_s, Wu_s, Wd_s, Wr):
    s = x2d.shape[0]
    return pl.pallas_call(
        _shared_router_body,
        grid=(s // _TM,),
        in_specs=[
            pl.BlockSpec((_TM, _H), lambda i: (i, 0)),
            pl.BlockSpec((_H, _I), lambda i: (0, 0)),
            pl.BlockSpec((_H, _I), lambda i: (0, 0)),
            pl.BlockSpec((_I, _H), lambda i: (0, 0)),
            pl.BlockSpec((_H, _E), lambda i: (0, 0)),
        ],
        out_specs=[
            pl.BlockSpec((_TM, _H), lambda i: (i, 0)),
            pl.BlockSpec((_K, _TM), lambda i: (0, i)),
            pl.BlockSpec((_K, _TM), lambda i: (0, i)),
            pl.BlockSpec((_TM, _HP), lambda i: (i, 0)),
        ],
        out_shape=[
            jax.ShapeDtypeStruct((s, _H), jnp.float32),
            jax.ShapeDtypeStruct((_K, s), jnp.int32),
            jax.ShapeDtypeStruct((_K, s), jnp.float32),
            jax.ShapeDtypeStruct((s, _HP), jnp.int32),
        ],
    )(x2d, Wg_s, Wu_s, Wd_s, Wr)


def _group_mlp_body(te_ref, tv_ref, xs_ref, wg_ref, wu_ref, wd_ref, out_ref):
    del te_ref
    t = pl.program_id(0)

    @pl.when(tv_ref[0, t] == 1)
    def _():
        xt = _unpack_bf16(xs_ref[...])
        g = jnp.dot(xt, wg_ref[0].astype(jnp.bfloat16),
                    preferred_element_type=jnp.float32)
        u = jnp.dot(xt, wu_ref[0].astype(jnp.bfloat16),
                    preferred_element_type=jnp.float32)
        h = (jax.nn.sigmoid(g) * u).astype(jnp.bfloat16)
        out_ref[...] = _pack_bf16(
            jnp.dot(h, wd_ref[0].astype(jnp.bfloat16),
                    preferred_element_type=jnp.float32))


def _group_mlp(xs, Wg, Wu, Wd, tile_eid, tile_valid):
    grid_spec = pltpu.PrefetchScalarGridSpec(
        num_scalar_prefetch=2,
        grid=(_TMAX,),
        in_specs=[
            pl.BlockSpec((_BT, _HP), lambda t, te, tv: (t * tv[0, t], 0)),
            pl.BlockSpec((1, _H, _I), lambda t, te, tv: (te[0, t], 0, 0)),
            pl.BlockSpec((1, _H, _I), lambda t, te, tv: (te[0, t], 0, 0)),
            pl.BlockSpec((1, _I, _H), lambda t, te, tv: (te[0, t], 0, 0)),
        ],
        out_specs=pl.BlockSpec(
            (_BT, _HP),
            lambda t, te, tv: (t * tv[0, t] + (1 - tv[0, t]) * _TMAX, 0)),
    )
    return pl.pallas_call(
        _group_mlp_body,
        grid_spec=grid_spec,
        out_shape=jax.ShapeDtypeStruct(((_TMAX + 1) * _BT, _HP), jnp.int32),
    )(tile_eid, tile_valid, xs, Wg, Wu, Wd)


def _sc_gather_rows(table, idx2, chunk):
    nw, b_per_w = idx2.shape
    b = nw * b_per_w
    d = table.shape[1]
    mesh = plsc.VectorSubcoreMesh(core_axis_name="c", subcore_axis_name="s",
                                  num_cores=2, num_subcores=16)

    @functools.partial(
        pl.kernel,
        out_type=jax.ShapeDtypeStruct((b, d), table.dtype),
        mesh=mesh,
        scratch_types=[
            pltpu.VMEM((chunk,), jnp.int32),
            pltpu.VMEM((chunk, d), table.dtype),
            pltpu.SemaphoreType.DMA,
        ],
    )
    def k(table_hbm, idx_hbm, out_hbm, idx_v, rows_v, sem):
        wid = lax.axis_index("s") * 2 + lax.axis_index("c")
        base = wid * b_per_w

        @pl.loop(0, b_per_w, step=chunk)
        def _(off):
            pltpu.sync_copy(idx_hbm.at[wid, pl.ds(off, chunk)], idx_v)
            pltpu.async_copy(table_hbm.at[idx_v], rows_v, sem).wait()
            pltpu.sync_copy(rows_v, out_hbm.at[pl.ds(base + off, chunk)])

    return k(table, idx2)


def _sc_scatter_rows(table, dst_idx2, out_rows, chunk):
    s, d = table.shape
    nw, b_per_w = dst_idx2.shape
    mesh = plsc.VectorSubcoreMesh(core_axis_name="c", subcore_axis_name="s",
                                  num_cores=2, num_subcores=16)

    @functools.partial(
        pl.kernel,
        out_type=jax.ShapeDtypeStruct((out_rows, d), table.dtype),
        mesh=mesh,
        scratch_types=[
            pltpu.VMEM((chunk,), jnp.int32),
            pltpu.VMEM((chunk, d), table.dtype),
            pltpu.SemaphoreType.DMA,
        ],
    )
    def k(table_hbm, idx_hbm, out_hbm, idx_v, rows_v, sem):
        wid = lax.axis_index("s") * 2 + lax.axis_index("c")
        base = wid * b_per_w

        @pl.loop(0, b_per_w, step=chunk)
        def _(off):
            src = lax.rem(base + off, s)
            pltpu.sync_copy(idx_hbm.at[wid, pl.ds(off, chunk)], idx_v)
            pltpu.sync_copy(table_hbm.at[pl.ds(src, chunk)], rows_v)
            pltpu.async_copy(rows_v, out_hbm.at[idx_v], sem).wait()

    return k(table, dst_idx2)


def _combine_body(base_ref, p0_ref, p1_ref, w_ref, out_ref):
    w0 = w_ref[0, 0, :][:, None]
    w1 = w_ref[1, 0, :][:, None]
    p0 = _unpack_bf16(p0_ref[...]).astype(jnp.float32)
    p1 = _unpack_bf16(p1_ref[...]).astype(jnp.float32)
    out_ref[...] = base_ref[...] + w0 * p0 + w1 * p1


def _combine(base, picked, w2s):
    s = base.shape[0]
    return pl.pallas_call(
        _combine_body,
        grid=(s // _TM,),
        in_specs=[
            pl.BlockSpec((_TM, _H), lambda i: (i, 0)),
            pl.BlockSpec((_TM, _HP), lambda i: (i, 0)),
            pl.BlockSpec((_TM, _HP), lambda i, _o=s // _TM: (_o + i, 0)),
            pl.BlockSpec((_K, 1, _TM), lambda i: (0, 0, i)),
        ],
        out_specs=pl.BlockSpec((_TM, _H), lambda i: (i, 0)),
        out_shape=jax.ShapeDtypeStruct((s, _H), jnp.float32),
    )(base, picked, picked, w2s.reshape(_K, 1, s))


def _plan_body(idx_ref, dest_ref, te_ref, tv_ref):
    k_, s_ = idx_ref.shape
    nb = (k_ * s_) // _BT
    eid = idx_ref[...]
    lane = lax.broadcasted_iota(jnp.int32, (k_, s_, _E), 2)
    oh3 = (eid[:, :, None] == lane).astype(jnp.float32).reshape(nb, _BT, _E)
    r_i = lax.broadcasted_iota(jnp.int32, (_BT, _BT), 0)
    c_i = lax.broadcasted_iota(jnp.int32, (_BT, _BT), 1)
    ltri = (c_i < r_i).astype(jnp.float32)
    intra = jnp.stack([
        jnp.dot(ltri, oh3[b], preferred_element_type=jnp.float32)
        for b in range(nb)
    ])
    btot = jnp.sum(oh3, axis=1)
    rb_i = lax.broadcasted_iota(jnp.int32, (nb, nb), 0)
    cb_i = lax.broadcasted_iota(jnp.int32, (nb, nb), 1)
    lb = (cb_i > rb_i).astype(jnp.float32)
    boff = jnp.dot(lb.T, btot, preferred_element_type=jnp.float32)
    counts = jnp.sum(btot, axis=0).reshape(1, _E)
    tiles = jnp.floor((counts + (_BT - 1.0)) * (1.0 / _BT))
    re_i = lax.broadcasted_iota(jnp.int32, (_E, _E), 0)
    ce_i = lax.broadcasted_iota(jnp.int32, (_E, _E), 1)
    su = (re_i < ce_i).astype(jnp.float32)
    ts = jnp.dot(tiles, su, preferred_element_type=jnp.float32)
    base_f = ts * float(_BT)
    rank3 = intra + boff[:, None, :] + base_f[0][None, None, :]
    dest_ref[...] = jnp.sum(oh3 * rank3, axis=2).astype(jnp.int32)
    ts_i = ts.astype(jnp.int32)
    t_i = lax.broadcasted_iota(jnp.int32, (128, _E), 0)
    te = jnp.sum((ts_i[0][None, :] <= t_i).astype(jnp.int32), axis=1) - 1
    te_ref[...] = te.reshape(1, 128)
    total = jnp.sum(tiles).astype(jnp.int32)
    tv = (lax.broadcasted_iota(jnp.int32, (1, 128), 1) < total)
    tv_ref[...] = tv.astype(jnp.int32)


def _plan(top_idx2s):
    n = top_idx2s.size
    nb = n // _BT
    dest, te, tv = pl.pallas_call(
        _plan_body,
        out_shape=[
            jax.ShapeDtypeStruct((nb, _BT), jnp.int32),
            jax.ShapeDtypeStruct((1, 128), jnp.int32),
            jax.ShapeDtypeStruct((1, 128), jnp.int32),
        ],
    )(top_idx2s)
    return te, tv, dest


def kernel(x, Wg_s, Wu_s, Wd_s, Wg, Wu, Wd, Wr):
    b, s, h = x.shape
    flat = x.reshape(s, h)
    base, top_idx2s, top_w2s, xpack = _shared_router(flat, Wg_s, Wu_s, Wd_s, Wr)
    tile_eid, tile_valid, dest = _plan(top_idx2s)
    xs = _sc_scatter_rows(xpack, dest, _TMAX * _BT, 64)
    ys = _group_mlp(xs, Wg, Wu, Wd, tile_eid, tile_valid)
    picked = _sc_gather_rows(ys, dest, 64)
    return _combine(base, picked, top_w2s * _SCALE).reshape(b, s, h)


# speedup vs baseline: 1.1158x; 1.0150x over previous
import functools

import jax
import jax.numpy as jnp
from jax import lax
from jax.experimental import pallas as pl
from jax.experimental.pallas import tpu as pltpu
from jax.experimental.pallas import tpu_sc as plsc

_H = 768
_I = 256
_E = 64
_K = 2
_SCALE = 1.0
_BT = 128
_TMAX = 96
_TM = 256
_HP = _H // 2


def _pack_bf16(a):
    ab = a.astype(jnp.bfloat16)
    lo = lax.convert_element_type(
        lax.bitcast_convert_type(ab[..., :_HP], jnp.int16), jnp.int32)
    hi = lax.convert_element_type(
        lax.bitcast_convert_type(ab[..., _HP:], jnp.int16), jnp.int32)
    return (lo & 0xFFFF) | (hi << 16)


def _unpack_bf16(p):
    lo = lax.bitcast_convert_type(
        lax.convert_element_type(p, jnp.int16), jnp.bfloat16)
    hi = lax.bitcast_convert_type(
        lax.convert_element_type(
            lax.shift_right_logical(p, 16), jnp.int16), jnp.bfloat16)
    return jnp.concatenate([lo, hi], axis=-1)


def _shared_router_body(x_ref, wg_ref, wu_ref, wd_ref, wr_ref,
                        base_ref, w_ref, xp_ref,
                        dest_ref, te_ref, tv_ref, idx_scr):
    i = pl.program_id(0)
    xt = x_ref[...]
    xb = xt.astype(jnp.bfloat16)
    g = jnp.dot(xb, wg_ref[...].astype(jnp.bfloat16),
                preferred_element_type=jnp.float32)
    u = jnp.dot(xb, wu_ref[...].astype(jnp.bfloat16),
                preferred_element_type=jnp.float32)
    h = (jax.nn.sigmoid(g) * u).astype(jnp.bfloat16)
    so = jnp.dot(h, wd_ref[...].astype(jnp.bfloat16),
                 preferred_element_type=jnp.float32)
    base_ref[...] = xt + so
    logits = jnp.dot(xt, wr_ref[...], preferred_element_type=jnp.float32)
    m = jnp.max(logits, axis=-1, keepdims=True)
    e = jnp.exp(logits - m)
    p = e / jnp.sum(e, axis=-1, keepdims=True)
    i1 = jnp.argmax(p, axis=-1).astype(jnp.int32)
    m1 = jnp.max(p, axis=-1)
    lane = lax.broadcasted_iota(jnp.int32, p.shape, 1)
    p2 = jnp.where(lane == i1[:, None], -1.0, p)
    i2 = jnp.argmax(p2, axis=-1).astype(jnp.int32)
    m2 = jnp.max(p2, axis=-1)
    idx_scr[:, pl.ds(i * _TM, _TM)] = jnp.stack([i1, i2], axis=0)
    w_ref[...] = jnp.stack([m1, m2], axis=0)
    xp_ref[...] = _pack_bf16(xb)

    @pl.when(i == pl.num_programs(0) - 1)
    def _():
        _plan_body(idx_scr, dest_ref, te_ref, tv_ref)


def _shared_router(x2d, Wg_s, Wu_s, Wd_s, Wr):
    s = x2d.shape[0]
    nb = (s * _K) // _BT
    return pl.pallas_call(
        _shared_router_body,
        grid=(s // _TM,),
        in_specs=[
            pl.BlockSpec((_TM, _H), lambda i: (i, 0)),
            pl.BlockSpec((_H, _I), lambda i: (0, 0)),
            pl.BlockSpec((_H, _I), lambda i: (0, 0)),
            pl.BlockSpec((_I, _H), lambda i: (0, 0)),
            pl.BlockSpec((_H, _E), lambda i: (0, 0)),
        ],
        out_specs=[
            pl.BlockSpec((_TM, _H), lambda i: (i, 0)),
            pl.BlockSpec((_K, _TM), lambda i: (0, i)),
            pl.BlockSpec((_TM, _HP), lambda i: (i, 0)),
            pl.BlockSpec((nb, _BT), lambda i: (0, 0)),
            pl.BlockSpec((1, 128), lambda i: (0, 0)),
            pl.BlockSpec((1, 128), lambda i: (0, 0)),
        ],
        out_shape=[
            jax.ShapeDtypeStruct((s, _H), jnp.float32),
            jax.ShapeDtypeStruct((_K, s), jnp.float32),
            jax.ShapeDtypeStruct((s, _HP), jnp.int32),
            jax.ShapeDtypeStruct((nb, _BT), jnp.int32),
            jax.ShapeDtypeStruct((1, 128), jnp.int32),
            jax.ShapeDtypeStruct((1, 128), jnp.int32),
        ],
        scratch_shapes=[pltpu.VMEM((_K, s), jnp.int32)],
    )(x2d, Wg_s, Wu_s, Wd_s, Wr)


def _group_mlp_body(te_ref, tv_ref, xs_ref, wg_ref, wu_ref, wd_ref, out_ref):
    del te_ref
    t = pl.program_id(0)

    @pl.when(tv_ref[0, t] == 1)
    def _():
        xt = _unpack_bf16(xs_ref[...])
        g = jnp.dot(xt, wg_ref[0].astype(jnp.bfloat16),
                    preferred_element_type=jnp.float32)
        u = jnp.dot(xt, wu_ref[0].astype(jnp.bfloat16),
                    preferred_element_type=jnp.float32)
        h = (jax.nn.sigmoid(g) * u).astype(jnp.bfloat16)
        out_ref[...] = _pack_bf16(
            jnp.dot(h, wd_ref[0].astype(jnp.bfloat16),
                    preferred_element_type=jnp.float32))


def _group_mlp(xs, Wg, Wu, Wd, tile_eid, tile_valid):
    grid_spec = pltpu.PrefetchScalarGridSpec(
        num_scalar_prefetch=2,
        grid=(_TMAX,),
        in_specs=[
            pl.BlockSpec((_BT, _HP), lambda t, te, tv: (t * tv[0, t], 0)),
            pl.BlockSpec((1, _H, _I), lambda t, te, tv: (te[0, t], 0, 0)),
            pl.BlockSpec((1, _H, _I), lambda t, te, tv: (te[0, t], 0, 0)),
            pl.BlockSpec((1, _I, _H), lambda t, te, tv: (te[0, t], 0, 0)),
        ],
        out_specs=pl.BlockSpec(
            (_BT, _HP),
            lambda t, te, tv: (t * tv[0, t] + (1 - tv[0, t]) * _TMAX, 0)),
    )
    return pl.pallas_call(
        _group_mlp_body,
        grid_spec=grid_spec,
        out_shape=jax.ShapeDtypeStruct(((_TMAX + 1) * _BT, _HP), jnp.int32),
    )(tile_eid, tile_valid, xs, Wg, Wu, Wd)


def _sc_gather_rows(table, idx2, chunk):
    nw, b_per_w = idx2.shape
    b = nw * b_per_w
    d = table.shape[1]
    mesh = plsc.VectorSubcoreMesh(core_axis_name="c", subcore_axis_name="s",
                                  num_cores=2, num_subcores=16)

    @functools.partial(
        pl.kernel,
        out_type=jax.ShapeDtypeStruct((b, d), table.dtype),
        mesh=mesh,
        scratch_types=[
            pltpu.VMEM((chunk,), jnp.int32),
            pltpu.VMEM((chunk, d), table.dtype),
            pltpu.SemaphoreType.DMA,
        ],
    )
    def k(table_hbm, idx_hbm, out_hbm, idx_v, rows_v, sem):
        wid = lax.axis_index("s") * 2 + lax.axis_index("c")
        base = wid * b_per_w

        @pl.loop(0, b_per_w, step=chunk)
        def _(off):
            pltpu.sync_copy(idx_hbm.at[wid, pl.ds(off, chunk)], idx_v)
            pltpu.async_copy(table_hbm.at[idx_v], rows_v, sem).wait()
            pltpu.sync_copy(rows_v, out_hbm.at[pl.ds(base + off, chunk)])

    return k(table, idx2)


def _sc_scatter_rows(table, dst_idx2, out_rows, chunk):
    s, d = table.shape
    nw, b_per_w = dst_idx2.shape
    mesh = plsc.VectorSubcoreMesh(core_axis_name="c", subcore_axis_name="s",
                                  num_cores=2, num_subcores=16)

    @functools.partial(
        pl.kernel,
        out_type=jax.ShapeDtypeStruct((out_rows, d), table.dtype),
        mesh=mesh,
        scratch_types=[
            pltpu.VMEM((chunk,), jnp.int32),
            pltpu.VMEM((chunk, d), table.dtype),
            pltpu.SemaphoreType.DMA,
        ],
    )
    def k(table_hbm, idx_hbm, out_hbm, idx_v, rows_v, sem):
        wid = lax.axis_index("s") * 2 + lax.axis_index("c")
        base = wid * b_per_w

        @pl.loop(0, b_per_w, step=chunk)
        def _(off):
            src = lax.rem(base + off, s)
            pltpu.sync_copy(idx_hbm.at[wid, pl.ds(off, chunk)], idx_v)
            pltpu.sync_copy(table_hbm.at[pl.ds(src, chunk)], rows_v)
            pltpu.async_copy(rows_v, out_hbm.at[idx_v], sem).wait()

    return k(table, dst_idx2)


def _combine_body(base_ref, p0_ref, p1_ref, w_ref, out_ref):
    w0 = w_ref[0, 0, :][:, None]
    w1 = w_ref[1, 0, :][:, None]
    p0 = _unpack_bf16(p0_ref[...]).astype(jnp.float32)
    p1 = _unpack_bf16(p1_ref[...]).astype(jnp.float32)
    out_ref[...] = base_ref[...] + w0 * p0 + w1 * p1


def _combine(base, picked, w2s):
    s = base.shape[0]
    return pl.pallas_call(
        _combine_body,
        grid=(s // _TM,),
        in_specs=[
            pl.BlockSpec((_TM, _H), lambda i: (i, 0)),
            pl.BlockSpec((_TM, _HP), lambda i: (i, 0)),
            pl.BlockSpec((_TM, _HP), lambda i, _o=s // _TM: (_o + i, 0)),
            pl.BlockSpec((_K, 1, _TM), lambda i: (0, 0, i)),
        ],
        out_specs=pl.BlockSpec((_TM, _H), lambda i: (i, 0)),
        out_shape=jax.ShapeDtypeStruct((s, _H), jnp.float32),
    )(base, picked, picked, w2s.reshape(_K, 1, s))


def _plan_body(idx_ref, dest_ref, te_ref, tv_ref):
    k_, s_ = idx_ref.shape
    nb = (k_ * s_) // _BT
    eid = idx_ref[...]
    lane = lax.broadcasted_iota(jnp.int32, (k_, s_, _E), 2)
    oh3 = (eid[:, :, None] == lane).astype(jnp.float32).reshape(nb, _BT, _E)
    r_i = lax.broadcasted_iota(jnp.int32, (_BT, _BT), 0)
    c_i = lax.broadcasted_iota(jnp.int32, (_BT, _BT), 1)
    ltri = (c_i < r_i).astype(jnp.float32)
    intra = jnp.stack([
        jnp.dot(ltri, oh3[b], preferred_element_type=jnp.float32)
        for b in range(nb)
    ])
    btot = jnp.sum(oh3, axis=1)
    rb_i = lax.broadcasted_iota(jnp.int32, (nb, nb), 0)
    cb_i = lax.broadcasted_iota(jnp.int32, (nb, nb), 1)
    lb = (cb_i > rb_i).astype(jnp.float32)
    boff = jnp.dot(lb.T, btot, preferred_element_type=jnp.float32)
    counts = jnp.sum(btot, axis=0).reshape(1, _E)
    tiles = jnp.floor((counts + (_BT - 1.0)) * (1.0 / _BT))
    re_i = lax.broadcasted_iota(jnp.int32, (_E, _E), 0)
    ce_i = lax.broadcasted_iota(jnp.int32, (_E, _E), 1)
    su = (re_i < ce_i).astype(jnp.float32)
    ts = jnp.dot(tiles, su, preferred_element_type=jnp.float32)
    base_f = ts * float(_BT)
    rank3 = intra + boff[:, None, :] + base_f[0][None, None, :]
    dest_ref[...] = jnp.sum(oh3 * rank3, axis=2).astype(jnp.int32)
    ts_i = ts.astype(jnp.int32)
    t_i = lax.broadcasted_iota(jnp.int32, (128, _E), 0)
    te = jnp.sum((ts_i[0][None, :] <= t_i).astype(jnp.int32), axis=1) - 1
    te_ref[...] = te.reshape(1, 128)
    total = jnp.sum(tiles).astype(jnp.int32)
    tv = (lax.broadcasted_iota(jnp.int32, (1, 128), 1) < total)
    tv_ref[...] = tv.astype(jnp.int32)


def kernel(x, Wg_s, Wu_s, Wd_s, Wg, Wu, Wd, Wr):
    b, s, h = x.shape
    flat = x.reshape(s, h)
    base, top_w2s, xpack, dest, tile_eid, tile_valid = _shared_router(
        flat, Wg_s, Wu_s, Wd_s, Wr)
    xs = _sc_scatter_rows(xpack, dest, _TMAX * _BT, 128)
    ys = _group_mlp(xs, Wg, Wu, Wd, tile_eid, tile_valid)
    picked = _sc_gather_rows(ys, dest, 128)
    return _combine(base, picked, top_w2s * _SCALE).reshape(b, s, h)
